# baseline (speedup 1.0000x reference)
import functools

import jax
import jax.numpy as jnp
from jax import lax
from jax.experimental import pallas as pl
from jax.experimental.pallas import tpu as pltpu
from jax.experimental.pallas import tpu_sc as plsc

N_NODES = 10000
N_EDGES = 320000
D_FEAT = 128
HIDDEN1 = 64
OUT_DIM = 16

NC = 2
NS = 16
NW = NC * NS
CHUNK = 40
NBUF = 6
K = 3


def _sc_aggregate(d):
    ept = N_EDGES // NW
    chunks = ept // CHUNK
    rows_a = 632
    rows_b = N_NODES - 15 * rows_a

    mesh = plsc.VectorSubcoreMesh(core_axis_name="c", subcore_axis_name="s",
                                  num_cores=NC, num_subcores=NS)

    @functools.partial(
        pl.kernel,
        out_type=jax.ShapeDtypeStruct((NC, N_NODES, d), jnp.float32),
        mesh=mesh,
        scratch_types=[
            pltpu.VMEM((ept,), jnp.int32),
            pltpu.VMEM((ept,), jnp.int32),
            pltpu.VMEM((NBUF * CHUNK, d), jnp.float32),
            pltpu.VMEM_SHARED((N_NODES, d), jnp.float32),
            pltpu.SemaphoreType.DMA((NBUF,)),
            pltpu.SemaphoreType.DMA((NBUF,)),
            pltpu.SemaphoreType.DMA((3,)),
        ],
    )
    def agg(x_hbm, src_hbm, dst_hbm, zero_hbm, out_hbm,
            src_v, dst_v, rows_v, acc_sh, gsem, ssem, isem):
        c = lax.axis_index("c")
        s = lax.axis_index("s")
        w = s * NC + c
        r0 = pl.multiple_of(s * rows_a, 8)
        pltpu.async_copy(src_hbm.at[w], src_v, isem.at[0])
        pltpu.async_copy(dst_hbm.at[w], dst_v, isem.at[1])

        @pl.when(s < NS - 1)
        def _():
            pltpu.async_copy(zero_hbm.at[pl.ds(r0, rows_a)],
                             acc_sh.at[pl.ds(r0, rows_a)], isem.at[2])

        @pl.when(s == NS - 1)
        def _():
            pltpu.async_copy(zero_hbm.at[pl.ds(15 * rows_a, rows_b)],
                             acc_sh.at[pl.ds(15 * rows_a, rows_b)],
                             isem.at[2])

        pltpu.make_async_copy(src_hbm.at[w], src_v, isem.at[0]).wait()

        def buf(b):
            return rows_v.at[pl.ds(b * CHUNK, CHUNK)]

        def gather_start(i, b):
            off = pl.multiple_of(i * CHUNK, 8)
            idx = src_v.at[pl.ds(off, CHUNK)]
            pltpu.async_copy(x_hbm.at[idx], buf(b), gsem.at[b])

        def gather_wait(i, b):
            off = pl.multiple_of(i * CHUNK, 8)
            idx = src_v.at[pl.ds(off, CHUNK)]
            pltpu.make_async_copy(x_hbm.at[idx], buf(b), gsem.at[b]).wait()

        def scatter_start(i, b):
            off = pl.multiple_of(i * CHUNK, 8)
            idx = dst_v.at[pl.ds(off, CHUNK)]
            pltpu.async_copy(buf(b), acc_sh.at[idx], ssem.at[b], add=True)

        def scatter_wait(i, b):
            off = pl.multiple_of(i * CHUNK, 8)
            idx = dst_v.at[pl.ds(off, CHUNK)]
            pltpu.make_async_copy(buf(b), acc_sh.at[idx], ssem.at[b]).wait()

        for i in range(NBUF - K):
            gather_start(i, i)

        pltpu.make_async_copy(dst_hbm.at[w], dst_v, isem.at[1]).wait()

        @pl.when(s < NS - 1)
        def _():
            pltpu.make_async_copy(zero_hbm.at[pl.ds(r0, rows_a)],
                                  acc_sh.at[pl.ds(r0, rows_a)],
                                  isem.at[2]).wait()

        @pl.when(s == NS - 1)
        def _():
            pltpu.make_async_copy(zero_hbm.at[pl.ds(15 * rows_a, rows_b)],
                                  acc_sh.at[pl.ds(15 * rows_a, rows_b)],
                                  isem.at[2]).wait()

        plsc.subcore_barrier()

        def chunk_body(i, _):
            b = lax.rem(i, NBUF)
            gather_wait(i, b)
            scatter_start(i, b)
            j = i + (NBUF - K)
            bj = lax.rem(j, NBUF)

            @pl.when(j < chunks)
            def _():
                @pl.when(j >= NBUF)
                def _():
                    scatter_wait(j - NBUF, bj)

                gather_start(j, bj)

            return 0

        lax.fori_loop(0, chunks, chunk_body, 0)
        for i in range(chunks - NBUF, chunks):
            scatter_wait(i, i % NBUF)
        plsc.subcore_barrier()

        @pl.when(s < NS - 1)
        def _():
            pltpu.sync_copy(acc_sh.at[pl.ds(r0, rows_a)],
                            out_hbm.at[c, pl.ds(r0, rows_a)])

        @pl.when(s == NS - 1)
        def _():
            pltpu.sync_copy(acc_sh.at[pl.ds(15 * rows_a, rows_b)],
                            out_hbm.at[c, pl.ds(15 * rows_a, rows_b)])

    return agg


def _tc_hidden(t2, w0a, w0b):
    blk = 1000
    grid = N_NODES // blk

    def body(t_ref, w0a_ref, w0b_ref, o_ref):
        t = t_ref[0] + t_ref[1]
        h0 = jnp.maximum(jnp.dot(t, w0a_ref[...],
                                 preferred_element_type=jnp.float32), 0.0)
        h1 = jnp.maximum(jnp.dot(t, w0b_ref[...],
                                 preferred_element_type=jnp.float32), 0.0)
        o_ref[...] = jnp.concatenate([h0, h1], axis=1)

    return pl.pallas_call(
        body,
        grid=(grid,),
        in_specs=[
            pl.BlockSpec((NC, blk, D_FEAT), lambda i: (0, i, 0)),
            pl.BlockSpec((D_FEAT, HIDDEN1), lambda i: (0, 0)),
            pl.BlockSpec((D_FEAT, HIDDEN1), lambda i: (0, 0)),
        ],
        out_specs=pl.BlockSpec((blk, 2 * HIDDEN1), lambda i: (i, 0)),
        out_shape=jax.ShapeDtypeStruct((N_NODES, 2 * HIDDEN1), jnp.float32),
    )(t2, w0a, w0b)


def _tc_out(u2, w1a, w1b, wm):
    blk = 1000
    grid = N_NODES // blk

    def body(wm_ref, u_ref, w1a_ref, w1b_ref, o_ref):
        u = u_ref[0] + u_ref[1]
        o_ref[...] = (
            jnp.dot(u[:, :HIDDEN1], w1a_ref[...],
                    preferred_element_type=jnp.float32) * wm_ref[0]
            + jnp.dot(u[:, HIDDEN1:], w1b_ref[...],
                      preferred_element_type=jnp.float32) * wm_ref[1]
        )

    return pl.pallas_call(
        body,
        grid=(grid,),
        in_specs=[
            pl.BlockSpec(memory_space=pltpu.SMEM),
            pl.BlockSpec((NC, blk, 2 * HIDDEN1), lambda i: (0, i, 0)),
            pl.BlockSpec((HIDDEN1, OUT_DIM), lambda i: (0, 0)),
            pl.BlockSpec((HIDDEN1, OUT_DIM), lambda i: (0, 0)),
        ],
        out_specs=pl.BlockSpec((blk, OUT_DIM), lambda i: (i, 0)),
        out_shape=jax.ShapeDtypeStruct((N_NODES, OUT_DIM), jnp.float32),
    )(wm, u2, w1a, w1b)


def kernel(x, edge_index, W0a, W1a, W0b, W1b, w_modal):
    ept = N_EDGES // NW
    src = edge_index[0].astype(jnp.int32).reshape(NW, ept)
    dst = edge_index[1].astype(jnp.int32).reshape(NW, ept)
    zeros128 = jnp.zeros((N_NODES, D_FEAT), jnp.float32)

    t2 = _sc_aggregate(D_FEAT)(x, src, dst, zeros128)
    h = _tc_hidden(t2, W0a, W0b)
    u2 = _sc_aggregate(D_FEAT)(h, src, dst, zeros128)
    return _tc_out(u2, W1a, W1b, w_modal)


# speedup vs baseline: 1.1172x; 1.1172x over previous
import functools

import jax
import jax.numpy as jnp
from jax import lax
from jax.experimental import pallas as pl
from jax.experimental.pallas import tpu as pltpu
from jax.experimental.pallas import tpu_sc as plsc

N_NODES = 10000
N_EDGES = 320000
D_FEAT = 128
HIDDEN1 = 64
OUT_DIM = 16

NC = 2
NS = 16
NW = NC * NS
CHUNK = 16
NBUF = 12
K = 4


def _sc_aggregate(d):
    ept = N_EDGES // NW
    chunks = ept // CHUNK
    rows_a = 632
    rows_b = N_NODES - 15 * rows_a

    mesh = plsc.VectorSubcoreMesh(core_axis_name="c", subcore_axis_name="s",
                                  num_cores=NC, num_subcores=NS)

    @functools.partial(
        pl.kernel,
        out_type=jax.ShapeDtypeStruct((NC, N_NODES, d), jnp.float32),
        mesh=mesh,
        scratch_types=[
            pltpu.VMEM((ept,), jnp.int32),
            pltpu.VMEM((ept,), jnp.int32),
            pltpu.VMEM((NBUF * CHUNK, d), jnp.float32),
            pltpu.VMEM_SHARED((N_NODES, d), jnp.float32),
            pltpu.SemaphoreType.DMA((NBUF,)),
            pltpu.SemaphoreType.DMA((NBUF,)),
            pltpu.SemaphoreType.DMA((3,)),
        ],
    )
    def agg(x_hbm, src_hbm, dst_hbm, zero_hbm, out_hbm,
            src_v, dst_v, rows_v, acc_sh, gsem, ssem, isem):
        c = lax.axis_index("c")
        s = lax.axis_index("s")
        w = s * NC + c
        r0 = pl.multiple_of(s * rows_a, 8)
        pltpu.async_copy(src_hbm.at[w], src_v, isem.at[0])
        pltpu.async_copy(dst_hbm.at[w], dst_v, isem.at[1])

        @pl.when(s < NS - 1)
        def _():
            pltpu.async_copy(zero_hbm.at[pl.ds(r0, rows_a)],
                             acc_sh.at[pl.ds(r0, rows_a)], isem.at[2])

        @pl.when(s == NS - 1)
        def _():
            pltpu.async_copy(zero_hbm.at[pl.ds(15 * rows_a, rows_b)],
                             acc_sh.at[pl.ds(15 * rows_a, rows_b)],
                             isem.at[2])

        pltpu.make_async_copy(src_hbm.at[w], src_v, isem.at[0]).wait()

        def buf(b):
            return rows_v.at[pl.ds(b * CHUNK, CHUNK)]

        def gather_start(i, b):
            off = pl.multiple_of(i * CHUNK, 8)
            idx = src_v.at[pl.ds(off, CHUNK)]
            pltpu.async_copy(x_hbm.at[idx], buf(b), gsem.at[b])

        def gather_wait(i, b):
            off = pl.multiple_of(i * CHUNK, 8)
            idx = src_v.at[pl.ds(off, CHUNK)]
            pltpu.make_async_copy(x_hbm.at[idx], buf(b), gsem.at[b]).wait()

        def scatter_start(i, b):
            off = pl.multiple_of(i * CHUNK, 8)
            idx = dst_v.at[pl.ds(off, CHUNK)]
            pltpu.async_copy(buf(b), acc_sh.at[idx], ssem.at[b], add=True)

        def scatter_wait(i, b):
            off = pl.multiple_of(i * CHUNK, 8)
            idx = dst_v.at[pl.ds(off, CHUNK)]
            pltpu.make_async_copy(buf(b), acc_sh.at[idx], ssem.at[b]).wait()

        for i in range(NBUF - K):
            gather_start(i, i)

        pltpu.make_async_copy(dst_hbm.at[w], dst_v, isem.at[1]).wait()

        @pl.when(s < NS - 1)
        def _():
            pltpu.make_async_copy(zero_hbm.at[pl.ds(r0, rows_a)],
                                  acc_sh.at[pl.ds(r0, rows_a)],
                                  isem.at[2]).wait()

        @pl.when(s == NS - 1)
        def _():
            pltpu.make_async_copy(zero_hbm.at[pl.ds(15 * rows_a, rows_b)],
                                  acc_sh.at[pl.ds(15 * rows_a, rows_b)],
                                  isem.at[2]).wait()

        plsc.subcore_barrier()

        def chunk_body(i, _):
            b = lax.rem(i, NBUF)
            gather_wait(i, b)
            scatter_start(i, b)
            j = i + (NBUF - K)
            bj = lax.rem(j, NBUF)

            @pl.when(j < chunks)
            def _():
                @pl.when(j >= NBUF)
                def _():
                    scatter_wait(j - NBUF, bj)

                gather_start(j, bj)

            return 0

        lax.fori_loop(0, chunks, chunk_body, 0)
        for i in range(chunks - NBUF, chunks):
            scatter_wait(i, i % NBUF)
        plsc.subcore_barrier()

        @pl.when(s < NS - 1)
        def _():
            pltpu.sync_copy(acc_sh.at[pl.ds(r0, rows_a)],
                            out_hbm.at[c, pl.ds(r0, rows_a)])

        @pl.when(s == NS - 1)
        def _():
            pltpu.sync_copy(acc_sh.at[pl.ds(15 * rows_a, rows_b)],
                            out_hbm.at[c, pl.ds(15 * rows_a, rows_b)])

    return agg


def _tc_hidden(t2, w0a, w0b):
    blk = 1000
    grid = N_NODES // blk

    def body(t_ref, w0a_ref, w0b_ref, o_ref):
        t = t_ref[0] + t_ref[1]
        h0 = jnp.maximum(jnp.dot(t, w0a_ref[...],
                                 preferred_element_type=jnp.float32), 0.0)
        h1 = jnp.maximum(jnp.dot(t, w0b_ref[...],
                                 preferred_element_type=jnp.float32), 0.0)
        o_ref[...] = jnp.concatenate([h0, h1], axis=1)

    return pl.pallas_call(
        body,
        grid=(grid,),
        in_specs=[
            pl.BlockSpec((NC, blk, D_FEAT), lambda i: (0, i, 0)),
            pl.BlockSpec((D_FEAT, HIDDEN1), lambda i: (0, 0)),
            pl.BlockSpec((D_FEAT, HIDDEN1), lambda i: (0, 0)),
        ],
        out_specs=pl.BlockSpec((blk, 2 * HIDDEN1), lambda i: (i, 0)),
        out_shape=jax.ShapeDtypeStruct((N_NODES, 2 * HIDDEN1), jnp.float32),
    )(t2, w0a, w0b)


def _tc_out(u2, w1a, w1b, wm):
    blk = 1000
    grid = N_NODES // blk

    def body(wm_ref, u_ref, w1a_ref, w1b_ref, o_ref):
        u = u_ref[0] + u_ref[1]
        o_ref[...] = (
            jnp.dot(u[:, :HIDDEN1], w1a_ref[...],
                    preferred_element_type=jnp.float32) * wm_ref[0]
            + jnp.dot(u[:, HIDDEN1:], w1b_ref[...],
                      preferred_element_type=jnp.float32) * wm_ref[1]
        )

    return pl.pallas_call(
        body,
        grid=(grid,),
        in_specs=[
            pl.BlockSpec(memory_space=pltpu.SMEM),
            pl.BlockSpec((NC, blk, 2 * HIDDEN1), lambda i: (0, i, 0)),
            pl.BlockSpec((HIDDEN1, OUT_DIM), lambda i: (0, 0)),
            pl.BlockSpec((HIDDEN1, OUT_DIM), lambda i: (0, 0)),
        ],
        out_specs=pl.BlockSpec((blk, OUT_DIM), lambda i: (i, 0)),
        out_shape=jax.ShapeDtypeStruct((N_NODES, OUT_DIM), jnp.float32),
    )(wm, u2, w1a, w1b)


def kernel(x, edge_index, W0a, W1a, W0b, W1b, w_modal):
    ept = N_EDGES // NW
    src = edge_index[0].astype(jnp.int32).reshape(NW, ept)
    dst = edge_index[1].astype(jnp.int32).reshape(NW, ept)
    zeros128 = jnp.zeros((N_NODES, D_FEAT), jnp.float32)

    t2 = _sc_aggregate(D_FEAT)(x, src, dst, zeros128)
    h = _tc_hidden(t2, W0a, W0b)
    u2 = _sc_aggregate(D_FEAT)(h, src, dst, zeros128)
    return _tc_out(u2, W1a, W1b, w_modal)


# speedup vs baseline: 1.1912x; 1.0662x over previous
import functools

import jax
import jax.numpy as jnp
from jax import lax
from jax.experimental import pallas as pl
from jax.experimental.pallas import tpu as pltpu
from jax.experimental.pallas import tpu_sc as plsc

N_NODES = 10000
N_EDGES = 320000
D_FEAT = 128
HIDDEN1 = 64
OUT_DIM = 16

NC = 2
NS = 16
NW = NC * NS
CHUNK = 40
NBUF = 6
K = 2


def _sc_aggregate(d):
    ept = N_EDGES // NW
    chunks = ept // CHUNK
    rows_a = 632
    rows_b = N_NODES - 15 * rows_a

    mesh = plsc.VectorSubcoreMesh(core_axis_name="c", subcore_axis_name="s",
                                  num_cores=NC, num_subcores=NS)

    @functools.partial(
        pl.kernel,
        out_type=jax.ShapeDtypeStruct((NC, N_NODES, d), jnp.float32),
        mesh=mesh,
        scratch_types=[
            pltpu.VMEM((ept,), jnp.int32),
            pltpu.VMEM((ept,), jnp.int32),
            pltpu.VMEM((NBUF * CHUNK, d), jnp.float32),
            pltpu.VMEM_SHARED((N_NODES, d), jnp.float32),
            pltpu.SemaphoreType.DMA((NBUF,)),
            pltpu.SemaphoreType.DMA((NBUF,)),
            pltpu.SemaphoreType.DMA((3,)),
        ],
    )
    def agg(x_hbm, src_hbm, dst_hbm, zero_hbm, out_hbm,
            src_v, dst_v, rows_v, acc_sh, gsem, ssem, isem):
        c = lax.axis_index("c")
        s = lax.axis_index("s")
        w = s * NC + c
        r0 = pl.multiple_of(s * rows_a, 8)
        pltpu.async_copy(src_hbm.at[w], src_v, isem.at[0])
        pltpu.async_copy(dst_hbm.at[w], dst_v, isem.at[1])

        @pl.when(s < NS - 1)
        def _():
            pltpu.async_copy(zero_hbm.at[pl.ds(r0, rows_a)],
                             acc_sh.at[pl.ds(r0, rows_a)], isem.at[2])

        @pl.when(s == NS - 1)
        def _():
            pltpu.async_copy(zero_hbm.at[pl.ds(15 * rows_a, rows_b)],
                             acc_sh.at[pl.ds(15 * rows_a, rows_b)],
                             isem.at[2])

        pltpu.make_async_copy(src_hbm.at[w], src_v, isem.at[0]).wait()

        def buf(b):
            return rows_v.at[pl.ds(b * CHUNK, CHUNK)]

        def gather_start(i, b):
            off = pl.multiple_of(i * CHUNK, 8)
            idx = src_v.at[pl.ds(off, CHUNK)]
            pltpu.async_copy(x_hbm.at[idx], buf(b), gsem.at[b])

        def gather_wait(i, b):
            off = pl.multiple_of(i * CHUNK, 8)
            idx = src_v.at[pl.ds(off, CHUNK)]
            pltpu.make_async_copy(x_hbm.at[idx], buf(b), gsem.at[b]).wait()

        def scatter_start(i, b):
            off = pl.multiple_of(i * CHUNK, 8)
            idx = dst_v.at[pl.ds(off, CHUNK)]
            pltpu.async_copy(buf(b), acc_sh.at[idx], ssem.at[b], add=True)

        def scatter_wait(i, b):
            off = pl.multiple_of(i * CHUNK, 8)
            idx = dst_v.at[pl.ds(off, CHUNK)]
            pltpu.make_async_copy(buf(b), acc_sh.at[idx], ssem.at[b]).wait()

        for i in range(NBUF - K):
            gather_start(i, i)

        pltpu.make_async_copy(dst_hbm.at[w], dst_v, isem.at[1]).wait()

        @pl.when(s < NS - 1)
        def _():
            pltpu.make_async_copy(zero_hbm.at[pl.ds(r0, rows_a)],
                                  acc_sh.at[pl.ds(r0, rows_a)],
                                  isem.at[2]).wait()

        @pl.when(s == NS - 1)
        def _():
            pltpu.make_async_copy(zero_hbm.at[pl.ds(15 * rows_a, rows_b)],
                                  acc_sh.at[pl.ds(15 * rows_a, rows_b)],
                                  isem.at[2]).wait()

        plsc.subcore_barrier()

        def chunk_body(i, _):
            b = lax.rem(i, NBUF)
            gather_wait(i, b)
            scatter_start(i, b)
            j = i + (NBUF - K)
            bj = lax.rem(j, NBUF)

            @pl.when(j < chunks)
            def _():
                @pl.when(j >= NBUF)
                def _():
                    scatter_wait(j - NBUF, bj)

                gather_start(j, bj)

            return 0

        lax.fori_loop(0, chunks, chunk_body, 0)
        for i in range(chunks - NBUF, chunks):
            scatter_wait(i, i % NBUF)
        plsc.subcore_barrier()

        @pl.when(s < NS - 1)
        def _():
            pltpu.sync_copy(acc_sh.at[pl.ds(r0, rows_a)],
                            out_hbm.at[c, pl.ds(r0, rows_a)])

        @pl.when(s == NS - 1)
        def _():
            pltpu.sync_copy(acc_sh.at[pl.ds(15 * rows_a, rows_b)],
                            out_hbm.at[c, pl.ds(15 * rows_a, rows_b)])

    return agg


def _tc_hidden(t2, w0a, w0b):
    blk = 2000
    grid = N_NODES // blk

    def body(t_ref, w0a_ref, w0b_ref, o_ref):
        t = t_ref[0] + t_ref[1]
        h0 = jnp.maximum(jnp.dot(t, w0a_ref[...],
                                 preferred_element_type=jnp.float32), 0.0)
        h1 = jnp.maximum(jnp.dot(t, w0b_ref[...],
                                 preferred_element_type=jnp.float32), 0.0)
        o_ref[...] = jnp.concatenate([h0, h1], axis=1)

    return pl.pallas_call(
        body,
        grid=(grid,),
        in_specs=[
            pl.BlockSpec((NC, blk, D_FEAT), lambda i: (0, i, 0)),
            pl.BlockSpec((D_FEAT, HIDDEN1), lambda i: (0, 0)),
            pl.BlockSpec((D_FEAT, HIDDEN1), lambda i: (0, 0)),
        ],
        out_specs=pl.BlockSpec((blk, 2 * HIDDEN1), lambda i: (i, 0)),
        out_shape=jax.ShapeDtypeStruct((N_NODES, 2 * HIDDEN1), jnp.float32),
    )(t2, w0a, w0b)


def _tc_out(u2, w1a, w1b, wm):
    blk = 2000
    grid = N_NODES // blk

    def body(wm_ref, u_ref, w1a_ref, w1b_ref, o_ref):
        u = u_ref[0] + u_ref[1]
        o_ref[...] = (
            jnp.dot(u[:, :HIDDEN1], w1a_ref[...],
                    preferred_element_type=jnp.float32) * wm_ref[0]
            + jnp.dot(u[:, HIDDEN1:], w1b_ref[...],
                      preferred_element_type=jnp.float32) * wm_ref[1]
        )

    return pl.pallas_call(
        body,
        grid=(grid,),
        in_specs=[
            pl.BlockSpec(memory_space=pltpu.SMEM),
            pl.BlockSpec((NC, blk, 2 * HIDDEN1), lambda i: (0, i, 0)),
            pl.BlockSpec((HIDDEN1, OUT_DIM), lambda i: (0, 0)),
            pl.BlockSpec((HIDDEN1, OUT_DIM), lambda i: (0, 0)),
        ],
        out_specs=pl.BlockSpec((blk, OUT_DIM), lambda i: (i, 0)),
        out_shape=jax.ShapeDtypeStruct((N_NODES, OUT_DIM), jnp.float32),
    )(wm, u2, w1a, w1b)


def kernel(x, edge_index, W0a, W1a, W0b, W1b, w_modal):
    ept = N_EDGES // NW
    src = edge_index[0].astype(jnp.int32).reshape(NW, ept)
    dst = edge_index[1].astype(jnp.int32).reshape(NW, ept)
    zeros128 = jnp.zeros((N_NODES, D_FEAT), jnp.float32)

    t2 = _sc_aggregate(D_FEAT)(x, src, dst, zeros128)
    h = _tc_hidden(t2, W0a, W0b)
    u2 = _sc_aggregate(D_FEAT)(h, src, dst, zeros128)
    return _tc_out(u2, W1a, W1b, w_modal)
